# baseline (speedup 1.0000x reference)
import functools

import jax
import jax.numpy as jnp
from jax import lax
from jax.experimental import pallas as pl
from jax.experimental.pallas import tpu as pltpu
from jax.experimental.pallas import tpu_sc as plsc

NC, NS, L = 2, 16, 16
NW = NC * NS
B, N, C1 = 16, 512, 81
OROW = 4 * C1
NH = N // 2
NV = NH // L
LCHUNKS = (16, 16, 16, 16, 17)
LSTARTS = (0, 16, 32, 48, 64)

_mesh = plsc.VectorSubcoreMesh(core_axis_name="c", subcore_axis_name="s")


@functools.partial(
    pl.kernel,
    out_type=(
        jax.ShapeDtypeStruct((NW, L), jnp.float32),
        jax.ShapeDtypeStruct((NW, L), jnp.float32),
    ),
    mesh=_mesh,
    compiler_params=pltpu.CompilerParams(
        use_tc_tiling_on_sc=True, needs_layout_passes=False
    ),
    scratch_types=[
        pltpu.VMEM((68, NH), jnp.float32),
        pltpu.VMEM((68, NH), jnp.float32),
        pltpu.VMEM((68, NH), jnp.float32),
        pltpu.VMEM((68, NH), jnp.float32),
        pltpu.VMEM((17, NH), jnp.float32),
        pltpu.VMEM((17, NH), jnp.float32),
        pltpu.VMEM((L,), jnp.float32),
        pltpu.SemaphoreType.DMA,
        pltpu.SemaphoreType.DMA,
    ],
)
def _sc_loss(o_hbm, t_hbm, l_hbm, num_hbm, den_hbm,
             o0, o1, t0, t1, l0, l1, stage, sem0, sem1):
    wid = lax.axis_index("s") * NC + lax.axis_index("c")
    b = wid // 2
    n0 = (wid % 2) * NH
    bufs = ((o0, t0, l0), (o1, t1, l1))
    sems = (sem0, sem1)

    def start(k):
        slot = k % 2
        l_lo, l_n = LSTARTS[k], LCHUNKS[k]
        ob, tb, lb = bufs[slot]
        return (
            pltpu.async_copy(
                o_hbm.at[pl.ds(4 * l_lo, 4 * l_n), b, pl.ds(n0, NH)],
                ob.at[pl.ds(0, 4 * l_n)], sems[slot]),
            pltpu.async_copy(
                t_hbm.at[pl.ds(4 * l_lo, 4 * l_n), b, pl.ds(n0, NH)],
                tb.at[pl.ds(0, 4 * l_n)], sems[slot]),
            pltpu.async_copy(
                l_hbm.at[pl.ds(l_lo, l_n), b, pl.ds(n0, NH)],
                lb.at[pl.ds(0, l_n)], sems[slot]),
        )

    num_acc = jnp.zeros((L,), jnp.float32)
    den_acc = jnp.zeros((L,), jnp.float32)

    descs = start(0)
    for k in range(len(LCHUNKS)):
        slot = k % 2
        if k + 1 < len(LCHUNKS):
            nxt = start(k + 1)
        for d in descs:
            d.wait()

        o_ref, t_ref, l_ref = bufs[slot]
        ls = LCHUNKS[k]

        def vec_body(v, carry, _k=k, _ls=ls, _o=o_ref, _t=t_ref, _l=l_ref):
            num, den = carry
            for li in range(_ls):
                if _k == 0 and li == 0:
                    continue
                lab = _l[li, pl.ds(v * L, L)]
                m = lab == 1.0
                den = den + lab
                for j in range(4):
                    o = _o[4 * li + j, pl.ds(v * L, L)]
                    t = _t[4 * li + j, pl.ds(v * L, L)]
                    d = o - t
                    ad = jnp.abs(d)
                    mn = jnp.minimum(ad, 1.0)
                    f = mn * (ad - 0.5 * mn)
                    num = num + jnp.where(m, f, 0.0)
            return num, den

        num_acc, den_acc = lax.fori_loop(0, NV, vec_body, (num_acc, den_acc))
        if k + 1 < len(LCHUNKS):
            descs = nxt

    stage[...] = num_acc
    pltpu.sync_copy(stage, num_hbm.at[wid])
    stage[...] = den_acc
    pltpu.sync_copy(stage, den_hbm.at[wid])


def kernel(output, target, labels_target):
    o = output.transpose(2, 0, 1)
    t = target.transpose(2, 0, 1)
    lt = labels_target.transpose(2, 0, 1)
    num, den = _sc_loss(o, t, lt)
    bsum = jnp.sum(den) + jnp.float32(0.0001 * B * N * (C1 - 1))
    return jnp.sum(num) / bsum


# speedup vs baseline: 1.7337x; 1.7337x over previous
import functools

import jax
import jax.numpy as jnp
from jax import lax
from jax.experimental import pallas as pl
from jax.experimental.pallas import tpu as pltpu
from jax.experimental.pallas import tpu_sc as plsc

NC, NS, L = 2, 16, 16
NW = NC * NS
B, N, C1 = 16, 512, 81
BH = B // 2
NV = N // L
UPT = 5

_mesh = plsc.VectorSubcoreMesh(core_axis_name="c", subcore_axis_name="s")

_plane = pltpu.VMEM((BH, N), jnp.float32)


@functools.partial(
    pl.kernel,
    out_type=jax.ShapeDtypeStruct((NW, 2, L), jnp.float32),
    mesh=_mesh,
    compiler_params=pltpu.CompilerParams(
        use_tc_tiling_on_sc=True,
        needs_layout_passes=False,
        disable_bounds_checks=True,
    ),
    scratch_types=[_plane] * 18 + [
        pltpu.VMEM((2, L), jnp.float32),
        pltpu.SemaphoreType.DMA,
        pltpu.SemaphoreType.DMA,
    ],
)
def _sc_loss(o_hbm, t_hbm, l_hbm, out_hbm, *refs):
    (o00, o01, o02, o03, o10, o11, o12, o13,
     t00, t01, t02, t03, t10, t11, t12, t13,
     lb0, lb1, stage, sem0, sem1) = refs
    obufs = ((o00, o01, o02, o03), (o10, o11, o12, o13))
    tbufs = ((t00, t01, t02, t03), (t10, t11, t12, t13))
    lbufs = (lb0, lb1)
    sems = (sem0, sem1)

    wid = lax.axis_index("s") * NC + lax.axis_index("c")
    u0 = wid * UPT

    def start(ui):
        slot = ui % 2
        u = u0 + ui
        g = 1 + u // 2
        b0 = (u % 2) * BH
        ds = []
        for j in range(4):
            ds.append(pltpu.async_copy(
                o_hbm.at[4 * g + j, pl.ds(b0, BH), :], obufs[slot][j], sems[slot]))
            ds.append(pltpu.async_copy(
                t_hbm.at[4 * g + j, pl.ds(b0, BH), :], tbufs[slot][j], sems[slot]))
        ds.append(pltpu.async_copy(
            l_hbm.at[g, pl.ds(b0, BH), :], lbufs[slot], sems[slot]))
        return ds

    num_acc = jnp.zeros((L,), jnp.float32)
    den_acc = jnp.zeros((L,), jnp.float32)

    descs = start(0)
    for ui in range(UPT):
        slot = ui % 2
        if ui + 1 < UPT:
            nxt = start(ui + 1)
        for d in descs:
            d.wait()

        obs, tbs, lb = obufs[slot], tbufs[slot], lbufs[slot]

        def b_body(bi, carry, _obs=obs, _tbs=tbs, _lb=lb):
            num, den = carry
            for v in range(NV):
                lab = _lb[bi, pl.ds(v * L, L)]
                m = lab == 1.0
                den = den + lab
                gacc = None
                for j in range(4):
                    o = _obs[j][bi, pl.ds(v * L, L)]
                    t = _tbs[j][bi, pl.ds(v * L, L)]
                    d = o - t
                    ad = jnp.abs(d)
                    mn = jnp.minimum(ad, 1.0)
                    f = mn * (ad - 0.5 * mn)
                    gacc = f if gacc is None else gacc + f
                num = num + jnp.where(m, gacc, 0.0)
            return num, den

        num_acc, den_acc = lax.fori_loop(0, BH, b_body, (num_acc, den_acc))
        if ui + 1 < UPT:
            descs = nxt

    stage[0] = num_acc
    stage[1] = den_acc
    pltpu.sync_copy(stage, out_hbm.at[wid])


def kernel(output, target, labels_target):
    o = output.transpose(2, 0, 1)
    t = target.transpose(2, 0, 1)
    lt = labels_target.transpose(2, 0, 1)
    part = _sc_loss(o, t, lt)
    s = jnp.sum(part, axis=(0, 2))
    return s[0] / (s[1] + jnp.float32(0.0001 * B * N * (C1 - 1)))
